# Initial kernel scaffold; baseline (speedup 1.0000x reference)
import jax
import jax.numpy as jnp
from jax.experimental import pallas as pl


def kernel(x, edge_index, edge_weight, cond, W_shared, b_shared, W_mu, b_mu, W_logstd, b_logstd):
    raise NotImplementedError("write your pallas kernel here")


# speedup vs baseline: 9.7517x; 9.7517x over previous
import functools

import jax
import jax.numpy as jnp
from jax import lax
from jax.experimental import pallas as pl
from jax.experimental.pallas import tpu as pltpu
from jax.experimental.pallas import tpu_sc as plsc

N = 10000
NP = 10240
D = 128
E = 320000
NCOND = 8
Z = 64

NT = 32
EPT = E // NT
CH = 80
NCHUNK = EPT // CH
NPT = NP // 16

RB = 1024
GRID = NP // RB


def _h0_body(x_ref, cond_ref, w1_ref, w2_ref, o_ref):
    c = cond_ref[0, 0, :]
    oh = (c[:, None] == lax.broadcasted_iota(jnp.int32, (RB, NCOND), 1)
          ).astype(jnp.float32)
    o_ref[...] = (jnp.dot(x_ref[...], w1_ref[...],
                          preferred_element_type=jnp.float32)
                  + jnp.dot(oh, w2_ref[...],
                            preferred_element_type=jnp.float32))


def _layer1_body(s1a_ref, s1b_ref, h0_ref, degt_ref, bsh_ref, wcat_ref, m_ref):
    deg = jnp.sum(degt_ref[0], axis=0) + 1.0
    dinv = jnp.where(deg > 0, 1.0 / deg, 0.0)
    h = jnp.maximum(
        dinv[:, None] * (s1a_ref[...] + s1b_ref[...] + h0_ref[...])
        + bsh_ref[...], 0.0)
    m_ref[...] = jnp.dot(h, wcat_ref[...], preferred_element_type=jnp.float32)


def _layer2_body(s2a_ref, s2b_ref, m_ref, degt_ref, bmu_ref, bls_ref,
                 mu_ref, ls_ref):
    deg = jnp.sum(degt_ref[0], axis=0) + 1.0
    dinv = jnp.where(deg > 0, 1.0 / deg, 0.0)
    out = dinv[:, None] * (s2a_ref[...] + s2b_ref[...] + m_ref[...])
    mu_ref[...] = out[:, :Z] + bmu_ref[...]
    ls_ref[...] = out[:, Z:] + bls_ref[...]


_row_spec = pl.BlockSpec((RB, D), lambda i: (i, 0))
_deg_spec = pl.BlockSpec((1, 2, RB), lambda i: (i, 0, 0))
_full = lambda shape: pl.BlockSpec(shape, lambda i: (0,) * len(shape))


def _sc_scatter_body(with_deg, row_hbm, col_hbm, ew_hbm, h_hbm, zrows_hbm,
                     zdeg_hbm, *rest):
    s_out, deg_out = rest[0], rest[1]
    acc_sh, row_idx, col_idx, ew_buf, rows_buf, deg_sh, sem = rest[2:]

    c = lax.axis_index("c")
    s = lax.axis_index("s")
    w = c * 16 + s

    pltpu.sync_copy(zrows_hbm, acc_sh.at[pl.ds(s * NPT, NPT)])
    if with_deg:
        pltpu.sync_copy(zdeg_hbm.at[pl.ds(s * NPT, NPT)],
                        deg_sh.at[pl.ds(s * NPT, NPT)])
    plsc.subcore_barrier()

    base = w * EPT

    def chunk(g, carry):
        off = base + g * CH
        pltpu.sync_copy(row_hbm.at[pl.ds(off, CH)], row_idx)
        pltpu.sync_copy(col_hbm.at[pl.ds(off, CH)], col_idx)
        pltpu.sync_copy(ew_hbm.at[pl.ds(off, CH)], ew_buf)
        pltpu.async_copy(h_hbm.at[row_idx], rows_buf, sem).wait()
        for g16 in range(CH // 16):
            wv = ew_buf[pl.ds(g16 * 16, 16)]
            for l in range(16):
                we = wv[l]
                e = g16 * 16 + l
                for j in range(D // 16):
                    sl = pl.ds(j * 16, 16)
                    rows_buf[e, sl] = rows_buf[e, sl] * we
        pltpu.sync_copy(rows_buf, acc_sh.at[col_idx], add=True)
        if with_deg:
            pltpu.sync_copy(ew_buf, deg_sh.at[col_idx], add=True)
        return carry

    lax.fori_loop(0, NCHUNK, chunk, 0)
    plsc.subcore_barrier()

    pltpu.sync_copy(acc_sh.at[pl.ds(s * NPT, NPT)],
                    s_out.at[c, pl.ds(s * NPT, NPT)])
    if with_deg:
        pltpu.sync_copy(deg_sh.at[pl.ds(s * NPT, NPT)],
                        deg_out.at[c, pl.ds(s * NPT, NPT)])


def _make_sc_kernel(with_deg):
    out_type = [jax.ShapeDtypeStruct((2, NP, D), jnp.float32),
                jax.ShapeDtypeStruct((2, NP), jnp.float32)]
    return functools.partial(
        pl.kernel,
        out_type=out_type,
        mesh=plsc.VectorSubcoreMesh(core_axis_name="c", subcore_axis_name="s"),
        scratch_types=[
            pltpu.VMEM_SHARED((NP, D), jnp.float32),
            pltpu.VMEM((CH,), jnp.int32),
            pltpu.VMEM((CH,), jnp.int32),
            pltpu.VMEM((CH,), jnp.float32),
            pltpu.VMEM((CH, D), jnp.float32),
            pltpu.VMEM_SHARED((NP,), jnp.float32),
            pltpu.SemaphoreType.DMA,
        ],
    )(functools.partial(_sc_scatter_body, with_deg))


@functools.cache
def _get_sc_kernel(with_deg):
    return _make_sc_kernel(with_deg)


def kernel(x, edge_index, edge_weight, cond, W_shared, b_shared,
           W_mu, b_mu, W_logstd, b_logstd):
    row = edge_index[0].astype(jnp.int32)
    col = edge_index[1].astype(jnp.int32)
    ew = edge_weight.astype(jnp.float32)
    xp = jnp.pad(x.astype(jnp.float32), ((0, NP - N), (0, 0)))
    cond3 = jnp.pad(cond.astype(jnp.int32), (0, NP - N)).reshape(GRID, 1, RB)
    W1 = W_shared[:D]
    W2 = W_shared[D:]
    Wcat = jnp.concatenate([W_mu, W_logstd], axis=1)
    bsh2 = b_shared.reshape(1, D)
    bmu2 = b_mu.reshape(1, Z)
    bls2 = b_logstd.reshape(1, Z)
    zrows = jnp.zeros((NPT, D), jnp.float32)
    zdeg = jnp.zeros((NP,), jnp.float32)

    h0 = pl.pallas_call(
        _h0_body,
        grid=(GRID,),
        in_specs=[_row_spec,
                  pl.BlockSpec((1, 1, RB), lambda i: (i, 0, 0)),
                  _full((D, D)), _full((NCOND, D))],
        out_specs=_row_spec,
        out_shape=jax.ShapeDtypeStruct((NP, D), jnp.float32),
    )(xp, cond3, W1, W2)

    s1, deg32 = _get_sc_kernel(True)(row, col, ew, h0, zrows, zdeg)
    degt = deg32.reshape(2, GRID, RB).transpose(1, 0, 2)

    m = pl.pallas_call(
        _layer1_body,
        grid=(GRID,),
        in_specs=[_row_spec, _row_spec, _row_spec, _deg_spec,
                  _full((1, D)), _full((D, D))],
        out_specs=_row_spec,
        out_shape=jax.ShapeDtypeStruct((NP, D), jnp.float32),
    )(s1[0], s1[1], h0, degt, bsh2, Wcat)

    s2, _ = _get_sc_kernel(False)(row, col, ew, m, zrows, zdeg)

    mu, logstd = pl.pallas_call(
        _layer2_body,
        grid=(GRID,),
        in_specs=[_row_spec, _row_spec, _row_spec, _deg_spec,
                  _full((1, Z)), _full((1, Z))],
        out_specs=[pl.BlockSpec((RB, Z), lambda i: (i, 0)),
                   pl.BlockSpec((RB, Z), lambda i: (i, 0))],
        out_shape=[jax.ShapeDtypeStruct((NP, Z), jnp.float32),
                   jax.ShapeDtypeStruct((NP, Z), jnp.float32)],
    )(s2[0], s2[1], m, degt, bmu2, bls2)

    return (mu[:N], logstd[:N])


# speedup vs baseline: 10.2547x; 1.0516x over previous
import functools

import jax
import jax.numpy as jnp
from jax import lax
from jax.experimental import pallas as pl
from jax.experimental.pallas import tpu as pltpu
from jax.experimental.pallas import tpu_sc as plsc

N = 10000
NP = 10240
D = 128
E = 320000
NCOND = 8
Z = 64

NT = 32
EPT = E // NT
CH = 125
CW = 128
NCHUNK = EPT // CH
SB = 10
NSB = NCHUNK // SB
NPT = NP // 16

RB = 1024
GRID = NP // RB


def _h0_body(x_ref, cond_ref, w1_ref, w2_ref, o_ref):
    c = cond_ref[0, 0, :]
    oh = (c[:, None] == lax.broadcasted_iota(jnp.int32, (RB, NCOND), 1)
          ).astype(jnp.float32)
    o_ref[...] = (jnp.dot(x_ref[...], w1_ref[...],
                          preferred_element_type=jnp.float32)
                  + jnp.dot(oh, w2_ref[...],
                            preferred_element_type=jnp.float32))


def _layer1_body(s1a_ref, s1b_ref, h0_ref, degt_ref, bsh_ref, wcat_ref, m_ref):
    deg = jnp.sum(degt_ref[0], axis=0) + 1.0
    dinv = jnp.where(deg > 0, 1.0 / deg, 0.0)
    h = jnp.maximum(
        dinv[:, None] * (s1a_ref[...] + s1b_ref[...] + h0_ref[...])
        + bsh_ref[...], 0.0)
    m_ref[...] = jnp.dot(h, wcat_ref[...], preferred_element_type=jnp.float32)


def _layer2_body(s2a_ref, s2b_ref, m_ref, degt_ref, bmu_ref, bls_ref,
                 mu_ref, ls_ref):
    deg = jnp.sum(degt_ref[0], axis=0) + 1.0
    dinv = jnp.where(deg > 0, 1.0 / deg, 0.0)
    out = dinv[:, None] * (s2a_ref[...] + s2b_ref[...] + m_ref[...])
    mu_ref[...] = out[:, :Z] + bmu_ref[...]
    ls_ref[...] = out[:, Z:] + bls_ref[...]


_row_spec = pl.BlockSpec((RB, D), lambda i: (i, 0))
_deg_spec = pl.BlockSpec((1, 2, RB), lambda i: (i, 0, 0))
_full = lambda shape: pl.BlockSpec(shape, lambda i: (0,) * len(shape))


def _sc_scatter_body(with_deg, rc_hbm, ew_hbm, h_hbm, zrows_hbm,
                     zdeg_hbm, *rest):
    s_out, deg_out = rest[0], rest[1]
    (acc_sh, deg_sh, rcm, ewf, rows0, rows1,
     gsem0, gsem1, ssem, dsem, msem) = rest[2:]
    rows = (rows0, rows1)
    gsem = (gsem0, gsem1)
    EWH = SB * CW

    c = lax.axis_index("c")
    s = lax.axis_index("s")
    w = c * 16 + s

    pltpu.sync_copy(rc_hbm.at[w, 0], rcm.at[0])
    pltpu.sync_copy(ew_hbm.at[w, 0], ewf.at[pl.ds(0, EWH)])
    pltpu.sync_copy(zrows_hbm, acc_sh.at[pl.ds(s * NPT, NPT)])
    if with_deg:
        pltpu.sync_copy(zdeg_hbm.at[pl.ds(s * NPT, NPT)],
                        deg_sh.at[pl.ds(s * NPT, NPT)])
    plsc.subcore_barrier()

    def scale(b, ew_base):
        for k16 in range(CW // 16):
            wv = ewf[pl.ds(ew_base + k16 * 16, 16)]
            for l in range(16):
                e = k16 * 16 + l
                we = wv[l]
                for j in range(D // 16):
                    sl = pl.ds(j * 16, 16)
                    rows[b][e, sl] = rows[b][e, sl] * we

    def wait_prev_streams():
        pltpu.make_async_copy(rows[0], acc_sh.at[rcm.at[0, 1, 0]],
                              ssem).wait()
        if with_deg:
            pltpu.make_async_copy(ewf.at[pl.ds(0, CW)],
                                  deg_sh.at[rcm.at[0, 1, 0]], dsem).wait()

    def stage(g, b):
        sb = g // SB
        k = g % SB
        msel = sb % 2
        msel_next = ((g + 1) // SB) % 2
        k_next = (g + 1) % SB

        @pl.when(g > 0)
        def _():
            wait_prev_streams()

        @pl.when((k == SB - 1) & (g + 1 < NCHUNK))
        def _():
            pltpu.make_async_copy(rc_hbm.at[w, 0], rcm.at[0], msem).wait()
            pltpu.make_async_copy(ew_hbm.at[w, 0], ewf.at[pl.ds(0, EWH)],
                                  msem).wait()

        @pl.when(g + 1 < NCHUNK)
        def _():
            pltpu.async_copy(h_hbm.at[rcm.at[msel_next, 0, k_next]],
                             rows[1 - b], gsem[1 - b])

        pltpu.make_async_copy(h_hbm.at[rcm.at[msel, 0, k]], rows[b],
                              gsem[b]).wait()
        scale(b, msel * EWH + k * CW)
        pltpu.async_copy(rows[b], acc_sh.at[rcm.at[msel, 1, k]], ssem,
                         add=True)
        if with_deg:
            pltpu.async_copy(ewf.at[pl.ds(msel * EWH + k * CW, CW)],
                             deg_sh.at[rcm.at[msel, 1, k]], dsem, add=True)

        @pl.when((k == 0) & (sb + 1 < NSB))
        def _():
            pltpu.async_copy(rc_hbm.at[w, sb + 1], rcm.at[1 - msel], msem)
            pltpu.async_copy(ew_hbm.at[w, sb + 1],
                             ewf.at[pl.ds((1 - msel) * EWH, EWH)], msem)

    pltpu.async_copy(h_hbm.at[rcm.at[0, 0, 0]], rows[0], gsem[0])

    def two_chunks(i, carry):
        stage(2 * i, 0)
        stage(2 * i + 1, 1)
        return carry

    lax.fori_loop(0, NCHUNK // 2, two_chunks, 0)

    wait_prev_streams()
    plsc.subcore_barrier()

    pltpu.sync_copy(acc_sh.at[pl.ds(s * NPT, NPT)],
                    s_out.at[c, pl.ds(s * NPT, NPT)])
    if with_deg:
        pltpu.sync_copy(deg_sh.at[pl.ds(s * NPT, NPT)],
                        deg_out.at[c, pl.ds(s * NPT, NPT)])


def _make_sc_kernel(with_deg):
    out_type = [jax.ShapeDtypeStruct((2, NP, D), jnp.float32),
                jax.ShapeDtypeStruct((2, NP), jnp.float32)]
    return functools.partial(
        pl.kernel,
        out_type=out_type,
        mesh=plsc.VectorSubcoreMesh(core_axis_name="c", subcore_axis_name="s"),
        scratch_types=[
            pltpu.VMEM_SHARED((NP, D), jnp.float32),
            pltpu.VMEM_SHARED((NP,), jnp.float32),
            pltpu.VMEM((2, 2, SB, CW), jnp.int32),
            pltpu.VMEM((2 * SB * CW,), jnp.float32),
            pltpu.VMEM((CW, D), jnp.float32),
            pltpu.VMEM((CW, D), jnp.float32),
            pltpu.SemaphoreType.DMA,
            pltpu.SemaphoreType.DMA,
            pltpu.SemaphoreType.DMA,
            pltpu.SemaphoreType.DMA,
            pltpu.SemaphoreType.DMA,
        ],
    )(functools.partial(_sc_scatter_body, with_deg))


@functools.cache
def _get_sc_kernel(with_deg):
    return _make_sc_kernel(with_deg)


def kernel(x, edge_index, edge_weight, cond, W_shared, b_shared,
           W_mu, b_mu, W_logstd, b_logstd):
    row = edge_index[0].astype(jnp.int32)
    col = edge_index[1].astype(jnp.int32)
    ew = edge_weight.astype(jnp.float32)
    xp = jnp.pad(x.astype(jnp.float32), ((0, NP - N), (0, 0)))
    cond3 = jnp.pad(cond.astype(jnp.int32), (0, NP - N)).reshape(GRID, 1, RB)
    W1 = W_shared[:D]
    W2 = W_shared[D:]
    Wcat = jnp.concatenate([W_mu, W_logstd], axis=1)
    bsh2 = b_shared.reshape(1, D)
    bmu2 = b_mu.reshape(1, Z)
    bls2 = b_logstd.reshape(1, Z)
    zrows = jnp.zeros((NPT, D), jnp.float32)
    zdeg = jnp.zeros((NP,), jnp.float32)

    h0 = pl.pallas_call(
        _h0_body,
        grid=(GRID,),
        in_specs=[_row_spec,
                  pl.BlockSpec((1, 1, RB), lambda i: (i, 0, 0)),
                  _full((D, D)), _full((NCOND, D))],
        out_specs=_row_spec,
        out_shape=jax.ShapeDtypeStruct((NP, D), jnp.float32),
    )(xp, cond3, W1, W2)

    pad4 = ((0, 0), (0, 0), (0, 0), (0, CW - CH))
    rowp = jnp.pad(row.reshape(NT, NSB, SB, CH), pad4)
    colp = jnp.pad(col.reshape(NT, NSB, SB, CH), pad4,
                   constant_values=NP - 1)
    rc4 = jnp.stack([rowp, colp], axis=2)
    ew4 = jnp.pad(ew.reshape(NT, NSB, SB, CH), pad4).reshape(NT, NSB, SB * CW)

    s1, deg32 = _get_sc_kernel(True)(rc4, ew4, h0, zrows, zdeg)
    degt = deg32.reshape(2, GRID, RB).transpose(1, 0, 2)

    m = pl.pallas_call(
        _layer1_body,
        grid=(GRID,),
        in_specs=[_row_spec, _row_spec, _row_spec, _deg_spec,
                  _full((1, D)), _full((D, D))],
        out_specs=_row_spec,
        out_shape=jax.ShapeDtypeStruct((NP, D), jnp.float32),
    )(s1[0], s1[1], h0, degt, bsh2, Wcat)

    s2, _ = _get_sc_kernel(False)(rc4, ew4, m, zrows, zdeg)

    mu, logstd = pl.pallas_call(
        _layer2_body,
        grid=(GRID,),
        in_specs=[_row_spec, _row_spec, _row_spec, _deg_spec,
                  _full((1, Z)), _full((1, Z))],
        out_specs=[pl.BlockSpec((RB, Z), lambda i: (i, 0)),
                   pl.BlockSpec((RB, Z), lambda i: (i, 0))],
        out_shape=[jax.ShapeDtypeStruct((NP, Z), jnp.float32),
                   jax.ShapeDtypeStruct((NP, Z), jnp.float32)],
    )(s2[0], s2[1], m, degt, bmu2, bls2)

    return (mu[:N], logstd[:N])


# speedup vs baseline: 10.2562x; 1.0002x over previous
import functools

import jax
import jax.numpy as jnp
from jax import lax
from jax.experimental import pallas as pl
from jax.experimental.pallas import tpu as pltpu
from jax.experimental.pallas import tpu_sc as plsc

N = 10000
NP = 10240
D = 128
E = 320000
NCOND = 8
Z = 64

NT = 32
EPT = E // NT
CH = 125
CW = 128
NCHUNK = EPT // CH
SB = 10
NSB = NCHUNK // SB
NPT = NP // 16

RB = 1024
GRID = NP // RB


def _h0_body(x_ref, cond_ref, w1_ref, w2_ref, o_ref):
    c = cond_ref[0, 0, :]
    oh = (c[:, None] == lax.broadcasted_iota(jnp.int32, (RB, NCOND), 1)
          ).astype(jnp.float32)
    o_ref[...] = (jnp.dot(x_ref[...], w1_ref[...],
                          preferred_element_type=jnp.float32)
                  + jnp.dot(oh, w2_ref[...],
                            preferred_element_type=jnp.float32))


def _layer1_body(s1a_ref, s1b_ref, h0_ref, degt_ref, bsh_ref, wcat_ref, m_ref):
    deg = jnp.sum(degt_ref[0], axis=0) + 1.0
    dinv = jnp.where(deg > 0, 1.0 / deg, 0.0)
    h = jnp.maximum(
        dinv[:, None] * (s1a_ref[...] + s1b_ref[...] + h0_ref[...])
        + bsh_ref[...], 0.0)
    m_ref[...] = jnp.dot(h, wcat_ref[...], preferred_element_type=jnp.float32)


def _layer2_body(s2a_ref, s2b_ref, m_ref, degt_ref, bmu_ref, bls_ref,
                 mu_ref, ls_ref):
    deg = jnp.sum(degt_ref[0], axis=0) + 1.0
    dinv = jnp.where(deg > 0, 1.0 / deg, 0.0)
    out = dinv[:, None] * (s2a_ref[...] + s2b_ref[...] + m_ref[...])
    mu_ref[...] = out[:, :Z] + bmu_ref[...]
    ls_ref[...] = out[:, Z:] + bls_ref[...]


_row_spec = pl.BlockSpec((RB, D), lambda i: (i, 0))
_deg_spec = pl.BlockSpec((1, 2, RB), lambda i: (i, 0, 0))
_full = lambda shape: pl.BlockSpec(shape, lambda i: (0,) * len(shape))


def _sc_scatter_body(with_deg, rc_hbm, ew_hbm, h_hbm, zrows_hbm,
                     zdeg_hbm, *rest):
    s_out, deg_out = rest[0], rest[1]
    (acc_sh, deg_sh, rcm, ewf, rows0, rows1,
     gsem0, gsem1, ssem, dsem, msem) = rest[2:]
    rows = (rows0, rows1)
    gsem = (gsem0, gsem1)
    EWH = SB * CW

    c = lax.axis_index("c")
    s = lax.axis_index("s")
    w = c * 16 + s

    pltpu.sync_copy(rc_hbm.at[w, 0], rcm.at[0])
    pltpu.sync_copy(ew_hbm.at[w, 0], ewf.at[pl.ds(0, EWH)])
    pltpu.sync_copy(zrows_hbm, acc_sh.at[pl.ds(s * NPT, NPT)])
    if with_deg:
        pltpu.sync_copy(zdeg_hbm.at[pl.ds(s * NPT, NPT)],
                        deg_sh.at[pl.ds(s * NPT, NPT)])
    plsc.subcore_barrier()

    def scale(b, ew_base):
        for k16 in range(CW // 16):
            wv = ewf[pl.ds(ew_base + k16 * 16, 16)]
            for l in range(16):
                e = k16 * 16 + l
                we = wv[l]
                for j in range(D // 16):
                    sl = pl.ds(j * 16, 16)
                    rows[b][e, sl] = rows[b][e, sl] * we

    def wait_prev_streams():
        pltpu.make_async_copy(rows[0], acc_sh.at[rcm.at[0, 1, 0]],
                              ssem).wait()
        if with_deg:
            pltpu.make_async_copy(ewf.at[pl.ds(0, CW)],
                                  deg_sh.at[rcm.at[0, 1, 0]], dsem).wait()

    def stage(g, b):
        sb = g // SB
        k = g % SB
        msel = sb % 2
        msel_next = ((g + 1) // SB) % 2
        k_next = (g + 1) % SB

        @pl.when(g > 0)
        def _():
            wait_prev_streams()

        @pl.when((k == SB - 1) & (g + 1 < NCHUNK))
        def _():
            pltpu.make_async_copy(rc_hbm.at[w, 0], rcm.at[0], msem).wait()
            pltpu.make_async_copy(ew_hbm.at[w, 0], ewf.at[pl.ds(0, EWH)],
                                  msem).wait()

        @pl.when(g + 1 < NCHUNK)
        def _():
            pltpu.async_copy(h_hbm.at[rcm.at[msel_next, 0, k_next,
                                             pl.ds(0, CW // 2)]],
                             rows[1 - b].at[pl.ds(0, CW // 2)], gsem[1 - b])
            pltpu.async_copy(h_hbm.at[rcm.at[msel_next, 0, k_next,
                                             pl.ds(CW // 2, CW // 2)]],
                             rows[1 - b].at[pl.ds(CW // 2, CW // 2)],
                             gsem[1 - b])

        for h in range(2):
            pltpu.make_async_copy(
                h_hbm.at[rcm.at[msel, 0, k, pl.ds(h * (CW // 2), CW // 2)]],
                rows[b].at[pl.ds(h * (CW // 2), CW // 2)], gsem[b]).wait()
        scale(b, msel * EWH + k * CW)
        pltpu.async_copy(rows[b], acc_sh.at[rcm.at[msel, 1, k]], ssem,
                         add=True)
        if with_deg:
            pltpu.async_copy(ewf.at[pl.ds(msel * EWH + k * CW, CW)],
                             deg_sh.at[rcm.at[msel, 1, k]], dsem, add=True)

        @pl.when((k == 0) & (sb + 1 < NSB))
        def _():
            pltpu.async_copy(rc_hbm.at[w, sb + 1], rcm.at[1 - msel], msem)
            pltpu.async_copy(ew_hbm.at[w, sb + 1],
                             ewf.at[pl.ds((1 - msel) * EWH, EWH)], msem)

    for h in range(2):
        pltpu.async_copy(
            h_hbm.at[rcm.at[0, 0, 0, pl.ds(h * (CW // 2), CW // 2)]],
            rows[0].at[pl.ds(h * (CW // 2), CW // 2)], gsem[0])

    def two_chunks(i, carry):
        stage(2 * i, 0)
        stage(2 * i + 1, 1)
        return carry

    lax.fori_loop(0, NCHUNK // 2, two_chunks, 0)

    wait_prev_streams()
    plsc.subcore_barrier()

    pltpu.sync_copy(acc_sh.at[pl.ds(s * NPT, NPT)],
                    s_out.at[c, pl.ds(s * NPT, NPT)])
    if with_deg:
        pltpu.sync_copy(deg_sh.at[pl.ds(s * NPT, NPT)],
                        deg_out.at[c, pl.ds(s * NPT, NPT)])


def _make_sc_kernel(with_deg):
    out_type = [jax.ShapeDtypeStruct((2, NP, D), jnp.float32),
                jax.ShapeDtypeStruct((2, NP), jnp.float32)]
    return functools.partial(
        pl.kernel,
        out_type=out_type,
        mesh=plsc.VectorSubcoreMesh(core_axis_name="c", subcore_axis_name="s"),
        scratch_types=[
            pltpu.VMEM_SHARED((NP, D), jnp.float32),
            pltpu.VMEM_SHARED((NP,), jnp.float32),
            pltpu.VMEM((2, 2, SB, CW), jnp.int32),
            pltpu.VMEM((2 * SB * CW,), jnp.float32),
            pltpu.VMEM((CW, D), jnp.float32),
            pltpu.VMEM((CW, D), jnp.float32),
            pltpu.SemaphoreType.DMA,
            pltpu.SemaphoreType.DMA,
            pltpu.SemaphoreType.DMA,
            pltpu.SemaphoreType.DMA,
            pltpu.SemaphoreType.DMA,
        ],
    )(functools.partial(_sc_scatter_body, with_deg))


@functools.cache
def _get_sc_kernel(with_deg):
    return _make_sc_kernel(with_deg)


def kernel(x, edge_index, edge_weight, cond, W_shared, b_shared,
           W_mu, b_mu, W_logstd, b_logstd):
    row = edge_index[0].astype(jnp.int32)
    col = edge_index[1].astype(jnp.int32)
    ew = edge_weight.astype(jnp.float32)
    xp = jnp.pad(x.astype(jnp.float32), ((0, NP - N), (0, 0)))
    cond3 = jnp.pad(cond.astype(jnp.int32), (0, NP - N)).reshape(GRID, 1, RB)
    W1 = W_shared[:D]
    W2 = W_shared[D:]
    Wcat = jnp.concatenate([W_mu, W_logstd], axis=1)
    bsh2 = b_shared.reshape(1, D)
    bmu2 = b_mu.reshape(1, Z)
    bls2 = b_logstd.reshape(1, Z)
    zrows = jnp.zeros((NPT, D), jnp.float32)
    zdeg = jnp.zeros((NP,), jnp.float32)

    h0 = pl.pallas_call(
        _h0_body,
        grid=(GRID,),
        in_specs=[_row_spec,
                  pl.BlockSpec((1, 1, RB), lambda i: (i, 0, 0)),
                  _full((D, D)), _full((NCOND, D))],
        out_specs=_row_spec,
        out_shape=jax.ShapeDtypeStruct((NP, D), jnp.float32),
    )(xp, cond3, W1, W2)

    pad4 = ((0, 0), (0, 0), (0, 0), (0, CW - CH))
    rowp = jnp.pad(row.reshape(NT, NSB, SB, CH), pad4)
    colp = jnp.pad(col.reshape(NT, NSB, SB, CH), pad4,
                   constant_values=NP - 1)
    rc4 = jnp.stack([rowp, colp], axis=2)
    ew4 = jnp.pad(ew.reshape(NT, NSB, SB, CH), pad4).reshape(NT, NSB, SB * CW)

    s1, deg32 = _get_sc_kernel(True)(rc4, ew4, h0, zrows, zdeg)
    degt = deg32.reshape(2, GRID, RB).transpose(1, 0, 2)

    m = pl.pallas_call(
        _layer1_body,
        grid=(GRID,),
        in_specs=[_row_spec, _row_spec, _row_spec, _deg_spec,
                  _full((1, D)), _full((D, D))],
        out_specs=_row_spec,
        out_shape=jax.ShapeDtypeStruct((NP, D), jnp.float32),
    )(s1[0], s1[1], h0, degt, bsh2, Wcat)

    s2, _ = _get_sc_kernel(False)(rc4, ew4, m, zrows, zdeg)

    mu, logstd = pl.pallas_call(
        _layer2_body,
        grid=(GRID,),
        in_specs=[_row_spec, _row_spec, _row_spec, _deg_spec,
                  _full((1, Z)), _full((1, Z))],
        out_specs=[pl.BlockSpec((RB, Z), lambda i: (i, 0)),
                   pl.BlockSpec((RB, Z), lambda i: (i, 0))],
        out_shape=[jax.ShapeDtypeStruct((NP, Z), jnp.float32),
                   jax.ShapeDtypeStruct((NP, Z), jnp.float32)],
    )(s2[0], s2[1], m, degt, bmu2, bls2)

    return (mu[:N], logstd[:N])


# speedup vs baseline: 15.5484x; 1.5160x over previous
import functools

import jax
import jax.numpy as jnp
from jax import lax
from jax.experimental import pallas as pl
from jax.experimental.pallas import tpu as pltpu
from jax.experimental.pallas import tpu_sc as plsc

N = 10000
NP = 10240
D = 128
E = 320000
NCOND = 8
Z = 64

NT = 32
EPT = E // NT
CH = 125
CW = 128
NCHUNK = EPT // CH
SB = 10
NSB = NCHUNK // SB
NPT = NP // 16

RB = 1024
GRID = NP // RB


def _pack_bf16_pair(h):
    bits = lax.bitcast_convert_type(h, jnp.int32) + jnp.int32(0x8000)
    lo = lax.shift_right_logical(bits[:, : D // 2], 16)
    hi = bits[:, D // 2:] & jnp.int32(-65536)
    return hi | lo


def _h0_body(x_ref, cond_ref, w1_ref, w2_ref, o_ref, ob_ref):
    c = cond_ref[0, 0, :]
    oh = (c[:, None] == lax.broadcasted_iota(jnp.int32, (RB, NCOND), 1)
          ).astype(jnp.float32)
    h = (jnp.dot(x_ref[...], w1_ref[...],
                 preferred_element_type=jnp.float32)
         + jnp.dot(oh, w2_ref[...],
                   preferred_element_type=jnp.float32))
    o_ref[...] = h
    ob_ref[...] = _pack_bf16_pair(h)


def _layer1_body(s1a_ref, s1b_ref, h0_ref, degt_ref, bsh_ref, wcat_ref,
                 m_ref, mb_ref):
    deg = degt_ref[0, :] + degt_ref[1, :] + 1.0
    dinv = jnp.where(deg > 0, 1.0 / deg, 0.0)
    h = jnp.maximum(
        dinv[:, None] * (s1a_ref[...] + s1b_ref[...] + h0_ref[...])
        + bsh_ref[...], 0.0)
    m = jnp.dot(h, wcat_ref[...], preferred_element_type=jnp.float32)
    m_ref[...] = m
    mb_ref[...] = _pack_bf16_pair(m)


def _layer2_body(s2a_ref, s2b_ref, m_ref, degt_ref, bmu_ref, bls_ref,
                 mu_ref, ls_ref):
    deg = degt_ref[0, :] + degt_ref[1, :] + 1.0
    dinv = jnp.where(deg > 0, 1.0 / deg, 0.0)
    out = dinv[:, None] * (s2a_ref[...] + s2b_ref[...] + m_ref[...])
    mu_ref[...] = out[:, :Z] + bmu_ref[...]
    ls_ref[...] = out[:, Z:] + bls_ref[...]


_row_spec = pl.BlockSpec((RB, D), lambda i: (i, 0))
_deg_spec = pl.BlockSpec((2, RB), lambda i: (0, i))
_full = lambda shape: pl.BlockSpec(shape, lambda i: (0,) * len(shape))


def _sc_scatter_body(with_deg, rc_hbm, ew_hbm, h_hbm, zrows_hbm,
                     zdeg_hbm, *rest):
    s_out, deg_out = rest[0], rest[1]
    (acc_sh, deg_sh, rcm, ewf, rows0, rows1, fbuf,
     gsem0, gsem1, ssem, dsem, msem) = rest[2:]
    rows = (rows0, rows1)
    gsem = (gsem0, gsem1)
    EWH = SB * CW

    c = lax.axis_index("c")
    s = lax.axis_index("s")
    w = c * 16 + s

    pltpu.sync_copy(rc_hbm.at[w, 0], rcm.at[0])
    pltpu.sync_copy(ew_hbm.at[w, 0], ewf.at[pl.ds(0, EWH)])
    pltpu.sync_copy(zrows_hbm, acc_sh.at[pl.ds(s * NPT, NPT)])
    if with_deg:
        pltpu.sync_copy(zdeg_hbm.at[pl.ds(s * NPT, NPT)],
                        deg_sh.at[pl.ds(s * NPT, NPT)])
    plsc.subcore_barrier()

    def scale(b, ew_base):
        for k16 in range(CW // 16):
            wv = ewf[pl.ds(ew_base + k16 * 16, 16)]
            for l in range(16):
                e = k16 * 16 + l
                we = wv[l]
                for q in range(D // 32):
                    pair = rows[b][e, pl.ds(q * 16, 16)]
                    lo = lax.bitcast_convert_type(pair << 16, jnp.float32)
                    hi = lax.bitcast_convert_type(pair & jnp.int32(-65536),
                                                  jnp.float32)
                    fbuf[e, pl.ds(q * 16, 16)] = lo * we
                    fbuf[e, pl.ds(D // 2 + q * 16, 16)] = hi * we

    def wait_prev_streams():
        pltpu.make_async_copy(fbuf, acc_sh.at[rcm.at[0, 1, 0]],
                              ssem).wait()
        if with_deg:
            pltpu.make_async_copy(ewf.at[pl.ds(0, CW)],
                                  deg_sh.at[rcm.at[0, 1, 0]], dsem).wait()

    def stage(g, b):
        sb = g // SB
        k = g % SB
        msel = sb % 2
        msel_next = ((g + 1) // SB) % 2
        k_next = (g + 1) % SB

        @pl.when(g > 0)
        def _():
            wait_prev_streams()

        @pl.when((k == SB - 1) & (g + 1 < NCHUNK))
        def _():
            pltpu.make_async_copy(rc_hbm.at[w, 0], rcm.at[0], msem).wait()
            pltpu.make_async_copy(ew_hbm.at[w, 0], ewf.at[pl.ds(0, EWH)],
                                  msem).wait()

        @pl.when(g + 1 < NCHUNK)
        def _():
            pltpu.async_copy(h_hbm.at[rcm.at[msel_next, 0, k_next]],
                             rows[1 - b], gsem[1 - b])

        pltpu.make_async_copy(h_hbm.at[rcm.at[msel, 0, k]], rows[b],
                              gsem[b]).wait()
        scale(b, msel * EWH + k * CW)
        pltpu.async_copy(fbuf, acc_sh.at[rcm.at[msel, 1, k]], ssem,
                         add=True)
        if with_deg:
            pltpu.async_copy(ewf.at[pl.ds(msel * EWH + k * CW, CW)],
                             deg_sh.at[rcm.at[msel, 1, k]], dsem, add=True)

        @pl.when((k == 0) & (sb + 1 < NSB))
        def _():
            pltpu.async_copy(rc_hbm.at[w, sb + 1], rcm.at[1 - msel], msem)
            pltpu.async_copy(ew_hbm.at[w, sb + 1],
                             ewf.at[pl.ds((1 - msel) * EWH, EWH)], msem)

    pltpu.async_copy(h_hbm.at[rcm.at[0, 0, 0]], rows[0], gsem[0])

    def two_chunks(i, carry):
        stage(2 * i, 0)
        stage(2 * i + 1, 1)
        return carry

    lax.fori_loop(0, NCHUNK // 2, two_chunks, 0)

    wait_prev_streams()
    plsc.subcore_barrier()

    pltpu.sync_copy(acc_sh.at[pl.ds(s * NPT, NPT)],
                    s_out.at[c, pl.ds(s * NPT, NPT)])
    if with_deg:
        pltpu.sync_copy(deg_sh.at[pl.ds(s * NPT, NPT)],
                        deg_out.at[c, pl.ds(s * NPT, NPT)])


def _make_sc_kernel(with_deg):
    out_type = [jax.ShapeDtypeStruct((2, NP, D), jnp.float32),
                jax.ShapeDtypeStruct((2, NP), jnp.float32)]
    return functools.partial(
        pl.kernel,
        out_type=out_type,
        mesh=plsc.VectorSubcoreMesh(core_axis_name="c", subcore_axis_name="s"),
        compiler_params=pltpu.CompilerParams(use_tc_tiling_on_sc=False),
        scratch_types=[
            pltpu.VMEM_SHARED((NP, D), jnp.float32),
            pltpu.VMEM_SHARED((NP,), jnp.float32),
            pltpu.VMEM((2, 2, SB, CW), jnp.int32),
            pltpu.VMEM((2 * SB * CW,), jnp.float32),
            pltpu.VMEM((CW, D // 2), jnp.int32),
            pltpu.VMEM((CW, D // 2), jnp.int32),
            pltpu.VMEM((CW, D), jnp.float32),
            pltpu.SemaphoreType.DMA,
            pltpu.SemaphoreType.DMA,
            pltpu.SemaphoreType.DMA,
            pltpu.SemaphoreType.DMA,
            pltpu.SemaphoreType.DMA,
        ],
    )(functools.partial(_sc_scatter_body, with_deg))


@functools.cache
def _get_sc_kernel(with_deg):
    return _make_sc_kernel(with_deg)


def kernel(x, edge_index, edge_weight, cond, W_shared, b_shared,
           W_mu, b_mu, W_logstd, b_logstd):
    row = edge_index[0].astype(jnp.int32)
    col = edge_index[1].astype(jnp.int32)
    ew = edge_weight.astype(jnp.float32)
    xp = jnp.pad(x.astype(jnp.float32), ((0, NP - N), (0, 0)))
    cond3 = jnp.pad(cond.astype(jnp.int32), (0, NP - N)).reshape(GRID, 1, RB)
    W1 = W_shared[:D]
    W2 = W_shared[D:]
    Wcat = jnp.concatenate([W_mu, W_logstd], axis=1)
    bsh2 = b_shared.reshape(1, D)
    bmu2 = b_mu.reshape(1, Z)
    bls2 = b_logstd.reshape(1, Z)
    zrows = jnp.zeros((NPT, D), jnp.float32)
    zdeg = jnp.zeros((NP,), jnp.float32)

    _half_spec = pl.BlockSpec((RB, D // 2), lambda i: (i, 0))
    h0, h0b = pl.pallas_call(
        _h0_body,
        grid=(GRID,),
        in_specs=[_row_spec,
                  pl.BlockSpec((1, 1, RB), lambda i: (i, 0, 0)),
                  _full((D, D)), _full((NCOND, D))],
        out_specs=[_row_spec, _half_spec],
        out_shape=[jax.ShapeDtypeStruct((NP, D), jnp.float32),
                   jax.ShapeDtypeStruct((NP, D // 2), jnp.int32)],
    )(xp, cond3, W1, W2)

    pad4 = ((0, 0), (0, 0), (0, 0), (0, CW - CH))
    rowp = jnp.pad(row.reshape(NT, NSB, SB, CH), pad4)
    colp = jnp.pad(col.reshape(NT, NSB, SB, CH), pad4,
                   constant_values=NP - 1)
    rc4 = jnp.stack([rowp, colp], axis=2)
    ew4 = jnp.pad(ew.reshape(NT, NSB, SB, CH), pad4).reshape(NT, NSB, SB * CW)

    s1, deg32 = _get_sc_kernel(True)(rc4, ew4, h0b, zrows, zdeg)
    degt = deg32

    m, mb = pl.pallas_call(
        _layer1_body,
        grid=(GRID,),
        in_specs=[_row_spec, _row_spec, _row_spec, _deg_spec,
                  _full((1, D)), _full((D, D))],
        out_specs=[_row_spec, _half_spec],
        out_shape=[jax.ShapeDtypeStruct((NP, D), jnp.float32),
                   jax.ShapeDtypeStruct((NP, D // 2), jnp.int32)],
    )(s1[0], s1[1], h0, degt, bsh2, Wcat)

    s2, _ = _get_sc_kernel(False)(rc4, ew4, mb, zrows, zdeg)

    mu, logstd = pl.pallas_call(
        _layer2_body,
        grid=(GRID,),
        in_specs=[_row_spec, _row_spec, _row_spec, _deg_spec,
                  _full((1, Z)), _full((1, Z))],
        out_specs=[pl.BlockSpec((RB, Z), lambda i: (i, 0)),
                   pl.BlockSpec((RB, Z), lambda i: (i, 0))],
        out_shape=[jax.ShapeDtypeStruct((NP, Z), jnp.float32),
                   jax.ShapeDtypeStruct((NP, Z), jnp.float32)],
    )(s2[0], s2[1], m, degt, bmu2, bls2)

    return (mu[:N], logstd[:N])
